# baseline (speedup 1.0000x reference)
import functools

import jax
import jax.numpy as jnp
from jax import lax
from jax.experimental import pallas as pl
from jax.experimental.pallas import tpu as pltpu
from jax.experimental.pallas import tpu_sc as plsc

_L = 16
_IW = 128


def _sc_mask_body(idx_hbm, mask_hbm, idx_v, val_v, zero_v, sem_z, sem_s,
                  *, n, m, nb):
    c = lax.axis_index("c")
    s = lax.axis_index("s")
    wid = s * 2 + c
    rows = idx_v.shape[0]
    bpw = nb // 32
    row0 = wid * rows

    pltpu.sync_copy(idx_hbm.at[pl.ds(row0, rows)], idx_v)

    def fill(j, _):
        zero_v[pl.ds(j * _L, _L)] = jnp.zeros((_L,), jnp.float32)
        return 0

    lax.fori_loop(0, n // _L, fill, 0)

    def gidx(j, _):
        r = j // (_IW // _L)
        k = j % (_IW // _L)
        b = (row0 + r) // (m // _IW)
        idx_v[r, pl.ds(k * _L, _L)] = idx_v[r, pl.ds(k * _L, _L)] + b * n
        val_v[r, pl.ds(k * _L, _L)] = jnp.ones((_L,), jnp.float32)
        return 0

    lax.fori_loop(0, rows * (_IW // _L), gidx, 0)

    zcopies = [
        pltpu.async_copy(zero_v, mask_hbm.at[pl.ds((wid * bpw + i) * n, n)],
                         sem_z)
        for i in range(bpw)
    ]
    for cp in zcopies:
        cp.wait()

    scopies = [
        pltpu.async_copy(val_v.at[j], mask_hbm.at[idx_v.at[j]], sem_s)
        for j in range(rows)
    ]
    for cp in scopies:
        cp.wait()


def _dense_body(m_ref, x_ref, o_ref, *, p):
    x = x_ref[...]
    mk = m_ref[...]
    lane = lax.broadcasted_iota(jnp.int32, (1, 1, 2 * p), 2)
    in_a = lane < p
    t = (lane % p).astype(jnp.float32) / (p - 1)
    s = jnp.where(in_a, x[:, :, 0:1], x[:, :, p:p + 1])
    e = jnp.where(in_a, x[:, :, p - 1:p], x[:, :, 2 * p - 1:2 * p])
    lerp = s + (e - s) * t
    m = jnp.where(in_a, mk[:, :, 0:1], mk[:, :, 1:2])
    o_ref[...] = jnp.where(m > 0.0, lerp, x)


def kernel(patches, masked_indices):
    B, N, P = patches.shape
    M = masked_indices.shape[1]
    idx2 = masked_indices.astype(jnp.int32).reshape(B * M // _IW, _IW)
    rows_per_worker = (B * M // _IW) // 32

    mesh = plsc.VectorSubcoreMesh(core_axis_name="c", subcore_axis_name="s")
    sc_mask = functools.partial(
        pl.kernel,
        mesh=mesh,
        out_type=jax.ShapeDtypeStruct((B * N,), jnp.float32),
        scratch_types=[
            pltpu.VMEM((rows_per_worker, _IW), jnp.int32),
            pltpu.VMEM((rows_per_worker, _IW), jnp.float32),
            pltpu.VMEM((N,), jnp.float32),
            pltpu.SemaphoreType.DMA,
            pltpu.SemaphoreType.DMA,
        ],
    )(functools.partial(_sc_mask_body, n=N, m=M, nb=B))
    mask = sc_mask(idx2)

    RB = N // 2
    BB = 4
    out = pl.pallas_call(
        functools.partial(_dense_body, p=P),
        grid=(B // BB,),
        in_specs=[
            pl.BlockSpec((BB, RB, 2), lambda b: (b, 0, 0)),
            pl.BlockSpec((BB, RB, 2 * P), lambda b: (b, 0, 0)),
        ],
        out_specs=pl.BlockSpec((BB, RB, 2 * P), lambda b: (b, 0, 0)),
        out_shape=jax.ShapeDtypeStruct((B, RB, 2 * P), patches.dtype),
    )(mask.reshape(B, RB, 2), patches.reshape(B, RB, 2 * P))
    return out.reshape(B, N, P)


# speedup vs baseline: 1.6012x; 1.6012x over previous
import functools

import jax
import jax.numpy as jnp
from jax import lax
from jax.experimental import pallas as pl


def _body(idx_ref, p1_ref, p2_ref, r1_ref, r2_ref, rm_ref, x_ref, o_ref, *, bb):
    p1 = p1_ref[...]
    p2 = p2_ref[...]
    r1 = r1_ref[...]
    r2 = r2_ref[...]
    rm = rm_ref[...]
    qi = lax.broadcasted_iota(jnp.int32, (64, 1), 0)
    for b in range(bb):
        idxr = idx_ref[b]
        hi = idxr // 64
        lo = idxr % 64
        a_t = (qi == hi).astype(jnp.float32)
        b2 = (qi == lo).astype(jnp.float32)
        cnt = lax.dot_general(a_t, b2, (((1,), (1,)), ((), ())),
                              preferred_element_type=jnp.float32)
        x = x_ref[b]
        starts = jnp.dot(x, p1, preferred_element_type=jnp.float32)
        ends = jnp.dot(x, p2, preferred_element_type=jnp.float32)
        lerp = (jnp.dot(starts, r1, preferred_element_type=jnp.float32)
                + jnp.dot(ends, r2, preferred_element_type=jnp.float32))
        mline = jnp.dot(cnt, rm, preferred_element_type=jnp.float32)
        o_ref[b] = jnp.where(mline > 0.0, lerp, x)


def kernel(patches, masked_indices):
    B, N, P = patches.shape
    M = masked_indices.shape[1]
    G = N // P
    W = N // G
    L = G * P
    idx3 = masked_indices.astype(jnp.int32).reshape(B, 1, M)

    li = jnp.arange(L, dtype=jnp.int32)[None, :]
    si = jnp.arange(W, dtype=jnp.int32)[:, None]
    sel = (li // P == si).astype(jnp.float32)
    t = (li % P).astype(jnp.float32) / (P - 1)
    r1 = sel * (1.0 - t)
    r2 = sel * t
    p1 = jnp.transpose((li == si * P).astype(jnp.float32))
    p2 = jnp.transpose((li == si * P + (P - 1)).astype(jnp.float32))

    BB = 4
    cspec = lambda shp: pl.BlockSpec(shp, lambda b: (0,) * len(shp))
    out = pl.pallas_call(
        functools.partial(_body, bb=BB),
        grid=(B // BB,),
        in_specs=[
            pl.BlockSpec((BB, 1, M), lambda b: (b, 0, 0)),
            cspec((L, W)), cspec((L, W)),
            cspec((W, L)), cspec((W, L)), cspec((W, L)),
            pl.BlockSpec((BB, G, L), lambda b: (b, 0, 0)),
        ],
        out_specs=pl.BlockSpec((BB, G, L), lambda b: (b, 0, 0)),
        out_shape=jax.ShapeDtypeStruct((B, G, L), patches.dtype),
    )(idx3, p1, p2, r1, r2, sel, patches.reshape(B, G, L))
    return out.reshape(B, N, P)
